# Initial kernel scaffold; baseline (speedup 1.0000x reference)
import jax
import jax.numpy as jnp
from jax.experimental import pallas as pl


def kernel(x, edge_index, edge_attr, batch, Win, bin_, gamma, beta, Wm, bm, Ws, bs, W1, b1, W2, b2):
    raise NotImplementedError("write your pallas kernel here")


# speedup vs baseline: 6.5180x; 6.5180x over previous
import functools

import jax
import jax.numpy as jnp
from jax import lax
from jax.experimental import pallas as pl
from jax.experimental.pallas import tpu as pltpu
from jax.experimental.pallas import tpu_sc as plsc

N = 10000
NPAD = 10240
E = 640000
NUM_GRAPHS = 64
D_IN = 128
D_EDGE = 16
H = 64
EMB = 128
L = 3

CHUNK = 128
NCHUNK = E // CHUNK
NTILE = 16
NW = 2 * NTILE
ROWS_PER_TILE = NPAD // NTILE


def _sc_edge_pre(edge_attr, dst, z16, z1, ones_c):
    mesh = plsc.VectorSubcoreMesh(core_axis_name="c", subcore_axis_name="s")

    @functools.partial(
        pl.kernel,
        mesh=mesh,
        out_type=[
            jax.ShapeDtypeStruct((2, NPAD, D_EDGE), jnp.float32),
            jax.ShapeDtypeStruct((2, NPAD), jnp.float32),
        ],
        scratch_types=[
            pltpu.VMEM((CHUNK,), jnp.int32),
            pltpu.VMEM((CHUNK, D_EDGE), jnp.float32),
            pltpu.VMEM((CHUNK,), jnp.float32),
            pltpu.VMEM_SHARED((NPAD, D_EDGE), jnp.float32),
            pltpu.VMEM_SHARED((NPAD,), jnp.float32),
        ],
        compiler_params=pltpu.CompilerParams(use_tc_tiling_on_sc=False),
    )
    def k(ea_hbm, dst_hbm, z16_hbm, z1_hbm, ones_hbm,
          ea_out, deg_out, idx_d, attr_v, ones_v, accE, accD):
        c = lax.axis_index("c")
        s = lax.axis_index("s")
        w = c * NTILE + s
        r0 = s * ROWS_PER_TILE
        pltpu.sync_copy(z16_hbm.at[pl.ds(r0, ROWS_PER_TILE)],
                        accE.at[pl.ds(r0, ROWS_PER_TILE)])
        pltpu.sync_copy(z1_hbm.at[pl.ds(r0, ROWS_PER_TILE)],
                        accD.at[pl.ds(r0, ROWS_PER_TILE)])
        pltpu.sync_copy(ones_hbm, ones_v)
        plsc.subcore_barrier()

        n_extra = NCHUNK % NW
        nw = (NCHUNK // NW) + jnp.where(w < n_extra, 1, 0)

        def step(kk, carry):
            base = (kk * NW + w) * CHUNK
            pltpu.sync_copy(dst_hbm.at[pl.ds(base, CHUNK)], idx_d)
            pltpu.sync_copy(ea_hbm.at[pl.ds(base, CHUNK)], attr_v)
            pltpu.sync_copy(attr_v, accE.at[idx_d], add=True)
            pltpu.sync_copy(ones_v, accD.at[idx_d], add=True)
            return carry

        lax.fori_loop(0, nw, step, 0)
        plsc.subcore_barrier()
        pltpu.sync_copy(accE.at[pl.ds(r0, ROWS_PER_TILE)],
                        ea_out.at[c, pl.ds(r0, ROWS_PER_TILE)])
        pltpu.sync_copy(accD.at[pl.ds(r0, ROWS_PER_TILE)],
                        deg_out.at[c, pl.ds(r0, ROWS_PER_TILE)])

    return k(edge_attr, dst, z16, z1, ones_c)


def _sc_agg(hn, src, dst, z64):
    mesh = plsc.VectorSubcoreMesh(core_axis_name="c", subcore_axis_name="s")

    @functools.partial(
        pl.kernel,
        mesh=mesh,
        out_type=jax.ShapeDtypeStruct((2, NPAD, H), jnp.float32),
        scratch_types=[
            pltpu.VMEM((CHUNK,), jnp.int32),
            pltpu.VMEM((CHUNK,), jnp.int32),
            pltpu.VMEM((CHUNK, H), jnp.float32),
            pltpu.VMEM_SHARED((NPAD, H), jnp.float32),
            pltpu.SemaphoreType.DMA,
        ],
        compiler_params=pltpu.CompilerParams(use_tc_tiling_on_sc=False),
    )
    def k(hn_hbm, src_hbm, dst_hbm, z_hbm, out_hbm, idx_s, idx_d, rows, acc, sem):
        c = lax.axis_index("c")
        s = lax.axis_index("s")
        w = c * NTILE + s
        r0 = s * ROWS_PER_TILE
        pltpu.sync_copy(z_hbm.at[pl.ds(r0, ROWS_PER_TILE)],
                        acc.at[pl.ds(r0, ROWS_PER_TILE)])
        plsc.subcore_barrier()

        n_extra = NCHUNK % NW
        nw = (NCHUNK // NW) + jnp.where(w < n_extra, 1, 0)

        def step(kk, carry):
            base = (kk * NW + w) * CHUNK
            pltpu.sync_copy(src_hbm.at[pl.ds(base, CHUNK)], idx_s)
            pltpu.sync_copy(dst_hbm.at[pl.ds(base, CHUNK)], idx_d)
            pltpu.async_copy(hn_hbm.at[idx_s], rows, sem).wait()
            pltpu.sync_copy(rows, acc.at[idx_d], add=True)
            return carry

        lax.fori_loop(0, nw, step, 0)
        plsc.subcore_barrier()
        pltpu.sync_copy(acc.at[pl.ds(r0, ROWS_PER_TILE)],
                        out_hbm.at[c, pl.ds(r0, ROWS_PER_TILE)])

    return k(hn, src, dst, z64)


def _inproj_body(x_ref, w_ref, b_ref, o_ref):
    o_ref[...] = jnp.dot(x_ref[...], w_ref[...],
                         preferred_element_type=jnp.float32) + b_ref[...]


def _pre_body(h_ref, g_ref, b_ref, ws_ref, bs_ref, eap_ref, wme_ref,
              bm_ref, degp_ref, hn_ref, base_ref):
    h = h_ref[...]
    mu = jnp.mean(h, axis=-1, keepdims=True)
    var = jnp.mean((h - mu) ** 2, axis=-1, keepdims=True)
    hn = (h - mu) / jnp.sqrt(var + 1e-5) * g_ref[...] + b_ref[...]
    hn_ref[...] = hn
    ea = eap_ref[0] + eap_ref[1]
    deg = degp_ref[0] + degp_ref[1]
    base = (jnp.dot(hn, ws_ref[...], preferred_element_type=jnp.float32)
            + bs_ref[...]
            + jnp.dot(ea, wme_ref[...], preferred_element_type=jnp.float32)
            + deg * bm_ref[...])
    base_ref[...] = base


def _post_body(h_ref, base_ref, gp_ref, wmh_ref, o_ref):
    g = gp_ref[0] + gp_ref[1]
    up = base_ref[...] + jnp.dot(g, wmh_ref[...],
                                 preferred_element_type=jnp.float32)
    o_ref[...] = h_ref[...] + jnp.maximum(up, 0.0)


def _pool_body(h_ref, batch_ref, w1_ref, b1_ref, w2_ref, b2_ref, o_ref):
    gids = lax.broadcasted_iota(jnp.int32, (1, NUM_GRAPHS), 1)
    onehot = (batch_ref[...] == gids).astype(jnp.float32)
    pooled = lax.dot_general(onehot, h_ref[...],
                             (((0,), (0,)), ((), ())),
                             preferred_element_type=jnp.float32)
    counts = lax.dot_general(onehot, jnp.ones((N, 1), jnp.float32),
                             (((0,), (0,)), ((), ())),
                             preferred_element_type=jnp.float32)
    pooled = pooled / jnp.maximum(counts, 1.0)
    e = jnp.maximum(
        jnp.dot(pooled, w1_ref[...], preferred_element_type=jnp.float32)
        + b1_ref[...], 0.0)
    e = jnp.dot(e, w2_ref[...], preferred_element_type=jnp.float32) + b2_ref[...]
    norm = jnp.sqrt(jnp.sum(e * e, axis=-1, keepdims=True))
    o_ref[...] = e / jnp.maximum(norm, 1e-12)


def _tc(body, out_shape, *args):
    return pl.pallas_call(body, out_shape=out_shape)(*args)


def kernel(x, edge_index, edge_attr, batch, Win, bin_, gamma, beta,
           Wm, bm, Ws, bs, W1, b1, W2, b2):
    f32 = jnp.float32
    src = edge_index[0]
    dst = edge_index[1]
    z64 = jnp.zeros((NPAD, H), f32)
    z16 = jnp.zeros((NPAD, D_EDGE), f32)
    z1 = jnp.zeros((NPAD,), f32)
    ones_c = jnp.ones((CHUNK,), f32)

    h = _tc(_inproj_body, jax.ShapeDtypeStruct((N, H), f32),
            x, Win, bin_.reshape(1, H))

    eap_pad, degp_pad = _sc_edge_pre(edge_attr, dst, z16, z1, ones_c)
    eap = eap_pad[:, :N]
    degp = degp_pad[:, :N].reshape(2, N, 1)

    for i in range(L):
        wm_h = Wm[i][:H]
        wm_e = Wm[i][H:]
        hn, base = _tc(
            _pre_body,
            [jax.ShapeDtypeStruct((N, H), f32),
             jax.ShapeDtypeStruct((N, H), f32)],
            h, gamma[i].reshape(1, H), beta[i].reshape(1, H),
            Ws[i], bs[i].reshape(1, H), eap, wm_e, bm[i].reshape(1, H), degp)
        gp = _sc_agg(hn, src, dst, z64)
        h = _tc(_post_body, jax.ShapeDtypeStruct((N, H), f32),
                h, base, gp[:, :N], wm_h)

    out = _tc(_pool_body, jax.ShapeDtypeStruct((NUM_GRAPHS, EMB), f32),
              h, batch.reshape(N, 1), W1, b1.reshape(1, EMB),
              W2, b2.reshape(1, EMB))
    return out


# speedup vs baseline: 8.7197x; 1.3378x over previous
import functools

import jax
import jax.numpy as jnp
from jax import lax
from jax.experimental import pallas as pl
from jax.experimental.pallas import tpu as pltpu
from jax.experimental.pallas import tpu_sc as plsc

N = 10000
NPAD = 10240
E = 640000
NUM_GRAPHS = 64
D_IN = 128
D_EDGE = 16
H = 64
EMB = 128
L = 3

CHUNK = 128
NTILE = 16
NW = 2 * NTILE
CPW = 158
PADE = NW * CPW * CHUNK
ROWS_PER_TILE = NPAD // NTILE

def _sc_params():
    return pltpu.CompilerParams(use_tc_tiling_on_sc=False)


def _sc_edge_pre(edge_attr, dst3, z16, z1, ones_c):
    mesh = plsc.VectorSubcoreMesh(core_axis_name="c", subcore_axis_name="s")

    @functools.partial(
        pl.kernel,
        mesh=mesh,
        out_type=[
            jax.ShapeDtypeStruct((2, NPAD, D_EDGE), jnp.float32),
            jax.ShapeDtypeStruct((2, NPAD), jnp.float32),
        ],
        scratch_types=[
            pltpu.VMEM((CPW, CHUNK), jnp.int32),
            pltpu.VMEM((CHUNK, D_EDGE), jnp.float32),
            pltpu.VMEM((CHUNK, D_EDGE), jnp.float32),
            pltpu.VMEM((CHUNK,), jnp.float32),
            pltpu.VMEM_SHARED((NPAD, D_EDGE), jnp.float32),
            pltpu.VMEM_SHARED((NPAD,), jnp.float32),
        ] + [pltpu.SemaphoreType.DMA] * 10,
        compiler_params=_sc_params(),
    )
    def k(ea_hbm, dst_hbm, z16_hbm, z1_hbm, ones_hbm, ea_out, deg_out,
          idxd, attr0, attr1, ones_v, accE, accD,
          semz0, semz1, semi, semo, sa0, sa1, se0, se1, sd0, sd1):
        c = lax.axis_index("c")
        s = lax.axis_index("s")
        w = c * NTILE + s
        r0 = s * ROWS_PER_TILE
        zc0 = pltpu.async_copy(z16_hbm.at[pl.ds(r0, ROWS_PER_TILE)],
                               accE.at[pl.ds(r0, ROWS_PER_TILE)], semz0)
        zc1 = pltpu.async_copy(z1_hbm.at[pl.ds(r0, ROWS_PER_TILE)],
                               accD.at[pl.ds(r0, ROWS_PER_TILE)], semz1)
        ic = pltpu.async_copy(dst_hbm.at[w], idxd, semi)
        oc = pltpu.async_copy(ones_hbm, ones_v, semo)
        zc0.wait()
        zc1.wait()
        plsc.subcore_barrier()
        ic.wait()
        oc.wait()

        def lbase(kk):
            b = (w * CPW + kk) * CHUNK
            return jnp.where(b < E, b, 0)

        def al(kk, buf, sem):
            pltpu.async_copy(ea_hbm.at[pl.ds(lbase(kk), CHUNK)], buf, sem)

        def alwait(kk, buf, sem):
            pltpu.make_async_copy(ea_hbm.at[pl.ds(lbase(kk), CHUNK)], buf,
                                  sem).wait()

        def fire(kk, buf, sem_e, sem_d):
            pltpu.async_copy(buf, accE.at[idxd.at[kk]], sem_e, add=True)
            pltpu.async_copy(ones_v, accD.at[idxd.at[kk]], sem_d, add=True)

        def ewait(kk, buf, sem):
            pltpu.make_async_copy(buf, accE.at[idxd.at[kk]], sem).wait()

        def dwait(kk, sem):
            pltpu.make_async_copy(ones_v, accD.at[idxd.at[kk]], sem).wait()

        al(0, attr0, sa0)
        al(1, attr1, sa1)

        def step(j, carry):
            k0 = 2 * j
            k1 = k0 + 1
            alwait(k0, attr0, sa0)
            fire(k0, attr0, se0, sd0)
            alwait(k1, attr1, sa1)
            fire(k1, attr1, se1, sd1)
            ewait(k0, attr0, se0)
            al(k0 + 2, attr0, sa0)
            ewait(k1, attr1, se1)
            al(k1 + 2, attr1, sa1)
            dwait(k0, sd0)
            dwait(k1, sd1)
            return carry

        lax.fori_loop(0, CPW // 2 - 1, step, 0)
        k0 = CPW - 2
        k1 = CPW - 1
        alwait(k0, attr0, sa0)
        fire(k0, attr0, se0, sd0)
        alwait(k1, attr1, sa1)
        fire(k1, attr1, se1, sd1)
        ewait(k0, attr0, se0)
        ewait(k1, attr1, se1)
        dwait(k0, sd0)
        dwait(k1, sd1)

        plsc.subcore_barrier()
        pltpu.sync_copy(accE.at[pl.ds(r0, ROWS_PER_TILE)],
                        ea_out.at[c, pl.ds(r0, ROWS_PER_TILE)])
        pltpu.sync_copy(accD.at[pl.ds(r0, ROWS_PER_TILE)],
                        deg_out.at[c, pl.ds(r0, ROWS_PER_TILE)])

    return k(edge_attr, dst3, z16, z1, ones_c)


def _sc_agg(hn, src3, dst3, z64):
    mesh = plsc.VectorSubcoreMesh(core_axis_name="c", subcore_axis_name="s")

    @functools.partial(
        pl.kernel,
        mesh=mesh,
        out_type=jax.ShapeDtypeStruct((2, NPAD, H), jnp.float32),
        scratch_types=[
            pltpu.VMEM((CPW, CHUNK), jnp.int32),
            pltpu.VMEM((CPW, CHUNK), jnp.int32),
            pltpu.VMEM((CHUNK, H), jnp.float32),
            pltpu.VMEM((CHUNK, H), jnp.float32),
            pltpu.VMEM_SHARED((NPAD, H), jnp.float32),
        ] + [pltpu.SemaphoreType.DMA] * 7,
        compiler_params=_sc_params(),
    )
    def k(hn_hbm, src_hbm, dst_hbm, z_hbm, out_hbm,
          idxs, idxd, rows0, rows1, acc,
          semz, semi0, semi1, sg0, sg1, ss0, ss1):
        c = lax.axis_index("c")
        s = lax.axis_index("s")
        w = c * NTILE + s
        r0 = s * ROWS_PER_TILE
        zc = pltpu.async_copy(z_hbm.at[pl.ds(r0, ROWS_PER_TILE)],
                              acc.at[pl.ds(r0, ROWS_PER_TILE)], semz)
        ic0 = pltpu.async_copy(src_hbm.at[w], idxs, semi0)
        ic1 = pltpu.async_copy(dst_hbm.at[w], idxd, semi1)
        zc.wait()
        plsc.subcore_barrier()
        ic0.wait()
        ic1.wait()

        def g(kk, buf, sem):
            pltpu.async_copy(hn_hbm.at[idxs.at[kk]], buf, sem)

        def gwait(kk, buf, sem):
            pltpu.make_async_copy(hn_hbm.at[idxs.at[kk]], buf, sem).wait()

        def sca(kk, buf, sem):
            pltpu.async_copy(buf, acc.at[idxd.at[kk]], sem, add=True)

        def swait(kk, buf, sem):
            pltpu.make_async_copy(buf, acc.at[idxd.at[kk]], sem).wait()

        g(0, rows0, sg0)
        g(1, rows1, sg1)

        def step(j, carry):
            k0 = 2 * j
            k1 = k0 + 1
            gwait(k0, rows0, sg0)
            sca(k0, rows0, ss0)
            gwait(k1, rows1, sg1)
            sca(k1, rows1, ss1)
            swait(k0, rows0, ss0)
            g(k0 + 2, rows0, sg0)
            swait(k1, rows1, ss1)
            g(k1 + 2, rows1, sg1)
            return carry

        lax.fori_loop(0, CPW // 2 - 1, step, 0)
        k0 = CPW - 2
        k1 = CPW - 1
        gwait(k0, rows0, sg0)
        sca(k0, rows0, ss0)
        gwait(k1, rows1, sg1)
        sca(k1, rows1, ss1)
        swait(k0, rows0, ss0)
        swait(k1, rows1, ss1)

        plsc.subcore_barrier()
        pltpu.sync_copy(acc.at[pl.ds(r0, ROWS_PER_TILE)],
                        out_hbm.at[c, pl.ds(r0, ROWS_PER_TILE)])

    return k(hn, src3, dst3, z64)


def _inproj_body(x_ref, w_ref, b_ref, o_ref):
    o_ref[...] = jnp.dot(x_ref[...], w_ref[...],
                         preferred_element_type=jnp.float32) + b_ref[...]


def _pre_body(h_ref, g_ref, b_ref, ws_ref, bs_ref, eap_ref, wme_ref,
              bm_ref, degp_ref, hn_ref, base_ref):
    h = h_ref[...]
    mu = jnp.mean(h, axis=-1, keepdims=True)
    var = jnp.mean((h - mu) ** 2, axis=-1, keepdims=True)
    hn = (h - mu) / jnp.sqrt(var + 1e-5) * g_ref[...] + b_ref[...]
    hn_ref[...] = hn
    ea = eap_ref[0, :N] + eap_ref[1, :N]
    deg = degp_ref[...]
    base = (jnp.dot(hn, ws_ref[...], preferred_element_type=jnp.float32)
            + bs_ref[...]
            + jnp.dot(ea, wme_ref[...], preferred_element_type=jnp.float32)
            + deg * bm_ref[...])
    base_ref[...] = base


def _post_body(h_ref, base_ref, gp_ref, wmh_ref, o_ref):
    g = gp_ref[0, :N] + gp_ref[1, :N]
    up = base_ref[...] + jnp.dot(g, wmh_ref[...],
                                 preferred_element_type=jnp.float32)
    o_ref[...] = h_ref[...] + jnp.maximum(up, 0.0)


def _pool_body(h_ref, batch_ref, w1_ref, b1_ref, w2_ref, b2_ref, o_ref):
    gids = lax.broadcasted_iota(jnp.int32, (1, NUM_GRAPHS), 1)
    onehot = (batch_ref[...] == gids).astype(jnp.float32)
    pooled = lax.dot_general(onehot, h_ref[...],
                             (((0,), (0,)), ((), ())),
                             preferred_element_type=jnp.float32)
    counts = lax.dot_general(onehot, jnp.ones((N, 1), jnp.float32),
                             (((0,), (0,)), ((), ())),
                             preferred_element_type=jnp.float32)
    pooled = pooled / jnp.maximum(counts, 1.0)
    e = jnp.maximum(
        jnp.dot(pooled, w1_ref[...], preferred_element_type=jnp.float32)
        + b1_ref[...], 0.0)
    e = jnp.dot(e, w2_ref[...], preferred_element_type=jnp.float32) + b2_ref[...]
    norm = jnp.sqrt(jnp.sum(e * e, axis=-1, keepdims=True))
    o_ref[...] = e / jnp.maximum(norm, 1e-12)


def _tc(body, out_shape, *args):
    return pl.pallas_call(body, out_shape=out_shape)(*args)


def kernel(x, edge_index, edge_attr, batch, Win, bin_, gamma, beta,
           Wm, bm, Ws, bs, W1, b1, W2, b2):
    f32 = jnp.float32
    src = edge_index[0]
    dst = edge_index[1]
    pad = PADE - E
    src3 = jnp.concatenate([src, jnp.zeros((pad,), jnp.int32)]
                           ).reshape(NW, CPW, CHUNK)
    dst3 = jnp.concatenate([dst, jnp.full((pad,), N, jnp.int32)]
                           ).reshape(NW, CPW, CHUNK)
    z64 = jnp.zeros((NPAD, H), f32)
    z16 = jnp.zeros((NPAD, D_EDGE), f32)
    z1 = jnp.zeros((NPAD,), f32)
    ones_c = jnp.ones((CHUNK,), f32)

    h = _tc(_inproj_body, jax.ShapeDtypeStruct((N, H), f32),
            x, Win, bin_.reshape(1, H))

    eap_pad, degp_pad = _sc_edge_pre(edge_attr, dst3, z16, z1, ones_c)
    degp = (degp_pad[0, :N] + degp_pad[1, :N]).reshape(N, 1)

    for i in range(L):
        wm_h = Wm[i][:H]
        wm_e = Wm[i][H:]
        hn, base = _tc(
            _pre_body,
            [jax.ShapeDtypeStruct((N, H), f32),
             jax.ShapeDtypeStruct((N, H), f32)],
            h, gamma[i].reshape(1, H), beta[i].reshape(1, H),
            Ws[i], bs[i].reshape(1, H), eap_pad, wm_e, bm[i].reshape(1, H),
            degp)
        gp = _sc_agg(hn, src3, dst3, z64)
        h = _tc(_post_body, jax.ShapeDtypeStruct((N, H), f32),
                h, base, gp, wm_h)

    out = _tc(_pool_body, jax.ShapeDtypeStruct((NUM_GRAPHS, EMB), f32),
              h, batch.reshape(N, 1), W1, b1.reshape(1, EMB),
              W2, b2.reshape(1, EMB))
    return out
